# Initial kernel scaffold; baseline (speedup 1.0000x reference)
import jax
import jax.numpy as jnp
from jax.experimental import pallas as pl


def kernel(x, edge_attr, u, We, be, Wn, bn, Wg, bg, edge_index, batch):
    raise NotImplementedError("write your pallas kernel here")


# speedup vs baseline: 1.5981x; 1.5981x over previous
import functools

import jax
import jax.numpy as jnp
from jax import lax
from jax.experimental import pallas as pl
from jax.experimental.pallas import tpu as pltpu
from jax.experimental.pallas import tpu_sc as plsc

_L = 16
_NC = 2
_NS = 16
_NW = _NC * _NS
_EB = 48


def _sc_edge_body(a_hbm, c_hbm, et_hbm, rowp, colp, ordp, ss_hbm, agg_hbm,
                  acc, a_buf, c_buf, et_buf, ridx, cidxg, cidxs, oidx, est_v,
                  sem_a, sem_c, sem_e):
    npt = acc.shape[0]
    d = acc.shape[1]
    nj = d // _L
    w = lax.axis_index("c") * _NS + lax.axis_index("s")

    pltpu.sync_copy(ss_hbm, est_v)
    ev = est_v[pl.ds(w, _L)]
    e_lo = ev[0]
    e_hi = ev[1]
    e0 = (e_lo // 8) * 8
    nblk = (e_hi - e0 + (_EB - 1)) // _EB

    zv = jnp.zeros((_L,), jnp.float32)

    def _zero(t, _):
        for j in range(nj):
            acc[t, pl.ds(j * _L, _L)] = zv
        return 0

    lax.fori_loop(0, npt, _zero, 0)

    w_base = w * npt

    def _block(k, _):
        e = e0 + k * _EB
        pltpu.sync_copy(rowp.at[pl.ds(e, _EB)], ridx)
        pltpu.sync_copy(colp.at[pl.ds(e, _EB)], cidxg)
        pltpu.sync_copy(colp.at[pl.ds(e, _EB + _L)], cidxs)
        pltpu.sync_copy(ordp.at[pl.ds(e, _EB)], oidx)
        cp_a = pltpu.async_copy(a_hbm.at[ridx], a_buf, sem_a)
        cp_c = pltpu.async_copy(c_hbm.at[cidxg], c_buf, sem_c)
        cp_e = pltpu.async_copy(et_hbm.at[oidx], et_buf, sem_e)
        cp_a.wait()
        cp_c.wait()
        cp_e.wait()

        def _edge(i, _):
            cv = cidxs[pl.ds(i, _L)]
            lcol = cv[0] - w_base

            @pl.when((lcol >= 0) & (lcol < npt))
            def _():
                for j in range(nj):
                    av = a_buf[i, pl.ds(j * _L, _L)]
                    bv = c_buf[i, pl.ds(j * _L, _L)]
                    tv = et_buf[i, pl.ds(j * _L, _L)]
                    v = jnp.maximum(av + bv + tv, 0.0)
                    plsc.addupdate(acc.at[lcol, pl.ds(j * _L, _L)], v)

            return 0

        lax.fori_loop(0, _EB, _edge, 0)
        return 0

    lax.fori_loop(0, nblk, _block, 0)

    pltpu.sync_copy(acc, agg_hbm.at[pl.ds(w * npt, npt)])


@functools.lru_cache(maxsize=None)
def _sc_edge_fn(npt, d):
    n2 = npt * _NW
    return pl.kernel(
        _sc_edge_body,
        out_type=jax.ShapeDtypeStruct((n2, d), jnp.float32),
        mesh=plsc.VectorSubcoreMesh(core_axis_name="c", subcore_axis_name="s"),
        scratch_types=[
            pltpu.VMEM((npt, d), jnp.float32),
            pltpu.VMEM((_EB, d), jnp.float32),
            pltpu.VMEM((_EB, d), jnp.float32),
            pltpu.VMEM((_EB, d), jnp.float32),
            pltpu.VMEM((_EB,), jnp.int32),
            pltpu.VMEM((_EB,), jnp.int32),
            pltpu.VMEM((_EB + _L,), jnp.int32),
            pltpu.VMEM((_EB,), jnp.int32),
            pltpu.VMEM((64,), jnp.int32),
            pltpu.SemaphoreType.DMA,
            pltpu.SemaphoreType.DMA,
            pltpu.SemaphoreType.DMA,
        ],
    )


def _sc_edge(a, c, et, rowp, colp, ordp, ss, npt, d):
    return _sc_edge_fn(npt, d)(a, c, et, rowp, colp, ordp, ss)


def _et_body(ea_ref, wee_ref, et_ref):
    et_ref[...] = jnp.dot(ea_ref[...], wee_ref[...],
                          preferred_element_type=jnp.float32)


def _stage_p_body(x_ref, oh_ref, u_ref, wea_ref, wec_ref, weu_ref, be_ref,
                  a_ref, c_ref):
    uw = jnp.dot(u_ref[...], weu_ref[...], preferred_element_type=jnp.float32)
    a_ref[...] = (jnp.dot(x_ref[...], wea_ref[...],
                          preferred_element_type=jnp.float32)
                  + jnp.dot(oh_ref[...], uw,
                            preferred_element_type=jnp.float32)
                  + be_ref[...])
    c_ref[...] = jnp.dot(x_ref[...], wec_ref[...],
                         preferred_element_type=jnp.float32)


def _stage_n_body(x_ref, agg_ref, oh_ref, u_ref, wnx_ref, wna_ref, wnu_ref,
                  bn_ref, wgu_ref, wgm_ref, bg_ref, xo_ref, uo_ref,
                  macc, cacc):
    i = pl.program_id(0)

    @pl.when(i == 0)
    def _():
        macc[...] = jnp.zeros_like(macc)
        cacc[...] = jnp.zeros_like(cacc)

    uw = jnp.dot(u_ref[...], wnu_ref[...], preferred_element_type=jnp.float32)
    xp = (jnp.dot(x_ref[...], wnx_ref[...],
                  preferred_element_type=jnp.float32)
          + jnp.dot(agg_ref[...], wna_ref[...],
                    preferred_element_type=jnp.float32)
          + jnp.dot(oh_ref[...], uw, preferred_element_type=jnp.float32)
          + bn_ref[...])
    xp = jnp.maximum(xp, 0.0)
    xo_ref[...] = xp
    oh = oh_ref[...]
    macc[...] += lax.dot_general(oh, xp, (((0,), (0,)), ((), ())),
                                 preferred_element_type=jnp.float32)
    ones = jnp.ones((oh.shape[0], 128), jnp.float32)
    cacc[...] += lax.dot_general(oh, ones, (((0,), (0,)), ((), ())),
                                 preferred_element_type=jnp.float32)

    @pl.when(i == pl.num_programs(0) - 1)
    def _():
        cnt = jnp.maximum(cacc[...][:, :1], 1.0)
        mean = macc[...] / cnt
        u2 = (jnp.dot(u_ref[...], wgu_ref[...],
                      preferred_element_type=jnp.float32)
              + jnp.dot(mean, wgm_ref[...],
                        preferred_element_type=jnp.float32)
              + bg_ref[...])
        uo_ref[...] = jnp.maximum(u2, 0.0)


def kernel(x, edge_attr, u, We, be, Wn, bn, Wg, bg, edge_index, batch):
    n, d = x.shape
    e, de = edge_attr.shape
    b = u.shape[0]
    npt = -(-n // (_NW * 8)) * 8
    n2 = npt * _NW

    row = edge_index[0]
    col = edge_index[1]

    order = jnp.argsort(col).astype(jnp.int32)
    col_s = jnp.take(col, order)
    row_s = jnp.take(row, order)
    bounds = jnp.arange(_NW + 1, dtype=jnp.int32) * npt
    ss = jnp.searchsorted(col_s, bounds).astype(jnp.int32)
    ss = jnp.pad(ss, (0, 64 - (_NW + 1)), constant_values=e)
    big = jnp.int32(1 << 22)
    pad = _EB + _L
    colp = jnp.concatenate([col_s, jnp.full((pad,), big, jnp.int32)])
    rowp = jnp.concatenate([row_s, jnp.zeros((pad,), jnp.int32)])
    ordp = jnp.concatenate([order, jnp.zeros((pad,), jnp.int32)])
    oh = (batch[:, None] == jnp.arange(b, dtype=batch.dtype)[None, :])
    oh = oh.astype(jnp.float32)

    wea = We[:d]
    wec = We[d:2 * d]
    wee = We[2 * d:2 * d + de]
    weu = We[2 * d + de:]
    wnx = Wn[:d]
    wna = Wn[d:2 * d]
    wnu = Wn[2 * d:]
    wgu = Wg[:d]
    wgm = Wg[d:]
    be2 = be.reshape(1, d)
    bn2 = bn.reshape(1, d)
    bg2 = bg.reshape(1, d)

    bep = 4000
    et = pl.pallas_call(
        _et_body,
        grid=(e // bep,),
        in_specs=[
            pl.BlockSpec((bep, de), lambda i: (i, 0)),
            pl.BlockSpec((de, d), lambda i: (0, 0)),
        ],
        out_specs=pl.BlockSpec((bep, d), lambda i: (i, 0)),
        out_shape=jax.ShapeDtypeStruct((e, d), jnp.float32),
    )(edge_attr, wee)

    bn_rows = 2000
    grid_n = n // bn_rows
    full = lambda i: (0, 0)

    stage_p = pl.pallas_call(
        _stage_p_body,
        grid=(grid_n,),
        in_specs=[
            pl.BlockSpec((bn_rows, d), lambda i: (i, 0)),
            pl.BlockSpec((bn_rows, b), lambda i: (i, 0)),
            pl.BlockSpec((b, d), full),
            pl.BlockSpec((d, d), full),
            pl.BlockSpec((d, d), full),
            pl.BlockSpec((d, d), full),
            pl.BlockSpec((1, d), full),
        ],
        out_specs=[
            pl.BlockSpec((bn_rows, d), lambda i: (i, 0)),
            pl.BlockSpec((bn_rows, d), lambda i: (i, 0)),
        ],
        out_shape=[
            jax.ShapeDtypeStruct((n, d), jnp.float32),
            jax.ShapeDtypeStruct((n, d), jnp.float32),
        ],
    )

    stage_n = pl.pallas_call(
        _stage_n_body,
        grid=(grid_n,),
        in_specs=[
            pl.BlockSpec((bn_rows, d), lambda i: (i, 0)),
            pl.BlockSpec((bn_rows, d), lambda i: (i, 0)),
            pl.BlockSpec((bn_rows, b), lambda i: (i, 0)),
            pl.BlockSpec((b, d), full),
            pl.BlockSpec((d, d), full),
            pl.BlockSpec((d, d), full),
            pl.BlockSpec((d, d), full),
            pl.BlockSpec((1, d), full),
            pl.BlockSpec((d, d), full),
            pl.BlockSpec((d, d), full),
            pl.BlockSpec((1, d), full),
        ],
        out_specs=[
            pl.BlockSpec((bn_rows, d), lambda i: (i, 0)),
            pl.BlockSpec((b, d), full),
        ],
        out_shape=[
            jax.ShapeDtypeStruct((n, d), jnp.float32),
            jax.ShapeDtypeStruct((b, d), jnp.float32),
        ],
        scratch_shapes=[
            pltpu.VMEM((b, d), jnp.float32),
            pltpu.VMEM((b, 128), jnp.float32),
        ],
    )

    outs = []
    steps = 4
    for _ in range(steps):
        a, c = stage_p(x, oh, u, wea, wec, weu, be2)
        agg = _sc_edge(a, c, et, rowp, colp, ordp, ss, npt, d)[:n]
        x, u = stage_n(x, agg, oh, u, wnx, wna, wnu, bn2, wgu, wgm, bg2)
        outs.append(u[:, None, :])
    return jnp.concatenate(outs, axis=1)


# speedup vs baseline: 1.8520x; 1.1589x over previous
import functools

import jax
import jax.numpy as jnp
from jax import lax
from jax.experimental import pallas as pl
from jax.experimental.pallas import tpu as pltpu
from jax.experimental.pallas import tpu_sc as plsc

_L = 16
_NC = 2
_NS = 16
_NW = _NC * _NS
_EB = 24


_RB = 96


def _sc_rows_body(tab_hbm, ordp_hbm, out_hbm, idxv, rows, sem):
    ept = out_hbm.shape[0] // _NW
    w = lax.axis_index("c") * _NS + lax.axis_index("s")
    base = w * ept

    def _blk(k, _):
        e = base + k * _RB
        pltpu.sync_copy(ordp_hbm.at[pl.ds(e, _RB)], idxv)
        pltpu.async_copy(tab_hbm.at[idxv], rows, sem).wait()
        pltpu.sync_copy(rows, out_hbm.at[pl.ds(e, _RB)])
        return 0

    lax.fori_loop(0, ept // _RB, _blk, 0)


@functools.lru_cache(maxsize=None)
def _sc_rows_fn(e2, de):
    return pl.kernel(
        _sc_rows_body,
        out_type=jax.ShapeDtypeStruct((e2, de), jnp.float32),
        mesh=plsc.VectorSubcoreMesh(core_axis_name="c", subcore_axis_name="s"),
        scratch_types=[
            pltpu.VMEM((_RB,), jnp.int32),
            pltpu.VMEM((_RB, de), jnp.float32),
            pltpu.SemaphoreType.DMA,
        ],
    )


def _sc_edge_body(a_hbm, c_hbm, et_hbm, rowp, colp, ss_hbm, agg_hbm,
                  acc,
                  a_buf0, c_buf0, et_buf0, ridx0, cidxg0, cidxs0,
                  a_buf1, c_buf1, et_buf1, ridx1, cidxg1, cidxs1,
                  est_v,
                  sem_a0, sem_c0, sem_e0, sem_a1, sem_c1, sem_e1):
    npt = acc.shape[0]
    d = acc.shape[1]
    nj = d // _L
    w = lax.axis_index("c") * _NS + lax.axis_index("s")

    bufs = (
        (a_buf0, c_buf0, et_buf0, ridx0, cidxg0, cidxs0,
         sem_a0, sem_c0, sem_e0),
        (a_buf1, c_buf1, et_buf1, ridx1, cidxg1, cidxs1,
         sem_a1, sem_c1, sem_e1),
    )

    pltpu.sync_copy(ss_hbm, est_v)
    ev = est_v[pl.ds(w, _L)]
    e_lo = ev[0]
    e_hi = ev[1]
    e0 = (e_lo // 8) * 8
    nblk = (e_hi - e0 + (_EB - 1)) // _EB

    w_base = w * npt

    def _start(k, b):
        a_buf, c_buf, et_buf, ridx, cidxg, cidxs, sa, sc_, se = bufs[b]
        e = e0 + k * _EB
        pltpu.sync_copy(rowp.at[pl.ds(e, _EB)], ridx)
        pltpu.sync_copy(colp.at[pl.ds(e, _EB)], cidxg)
        pltpu.sync_copy(colp.at[pl.ds(e, _EB + _L)], cidxs)
        pltpu.async_copy(a_hbm.at[ridx], a_buf, sa)
        pltpu.async_copy(c_hbm.at[cidxg], c_buf, sc_)
        pltpu.async_copy(et_hbm.at[pl.ds(e, _EB)], et_buf, se)

    def _finish(k, b):
        a_buf, c_buf, et_buf, ridx, cidxg, cidxs, sa, sc_, se = bufs[b]
        e = e0 + k * _EB
        pltpu.make_async_copy(a_hbm.at[ridx], a_buf, sa).wait()
        pltpu.make_async_copy(c_hbm.at[cidxg], c_buf, sc_).wait()
        pltpu.make_async_copy(et_hbm.at[pl.ds(e, _EB)], et_buf, se).wait()

    def _compute(b):
        a_buf, c_buf, et_buf, ridx, cidxg, cidxs, sa, sc_, se = bufs[b]

        @plsc.parallel_loop(0, _EB, 1, unroll=4)
        def _edge(i):
            cv = cidxs[pl.ds(i, _L)]
            lcol = cv[0] - w_base

            @pl.when((lcol >= 0) & (lcol < npt))
            def _():
                for j in range(nj):
                    av = a_buf[i, pl.ds(j * _L, _L)]
                    bv = c_buf[i, pl.ds(j * _L, _L)]
                    tv = et_buf[i, pl.ds(j * _L, _L)]
                    v = jnp.maximum(av + bv + tv, 0.0)
                    plsc.addupdate(acc.at[lcol, pl.ds(j * _L, _L)], v)

    for b in range(2):
        @pl.when(b < nblk)
        def _():
            _start(b, b)

    zv = jnp.zeros((_L,), jnp.float32)

    def _zero(t, _):
        for j in range(nj):
            acc[t, pl.ds(j * _L, _L)] = zv
        return 0

    lax.fori_loop(0, npt, _zero, 0)

    def _outer(k2, _):
        for b in range(2):
            k = k2 * 2 + b

            @pl.when(k < nblk)
            def _():
                _finish(k, b)
                _compute(b)

                @pl.when(k + 2 < nblk)
                def _():
                    _start(k + 2, b)

        return 0

    lax.fori_loop(0, (nblk + 1) // 2, _outer, 0)

    pltpu.sync_copy(acc, agg_hbm.at[pl.ds(w * npt, npt)])


@functools.lru_cache(maxsize=None)
def _sc_edge_fn(npt, d):
    n2 = npt * _NW
    set_scratch = [
        pltpu.VMEM((_EB, d), jnp.float32),
        pltpu.VMEM((_EB, d), jnp.float32),
        pltpu.VMEM((_EB, d), jnp.float32),
        pltpu.VMEM((_EB,), jnp.int32),
        pltpu.VMEM((_EB,), jnp.int32),
        pltpu.VMEM((_EB + _L,), jnp.int32),
    ]
    return pl.kernel(
        _sc_edge_body,
        out_type=jax.ShapeDtypeStruct((n2, d), jnp.float32),
        mesh=plsc.VectorSubcoreMesh(core_axis_name="c", subcore_axis_name="s"),
        scratch_types=(
            [pltpu.VMEM((npt, d), jnp.float32)]
            + set_scratch + set_scratch
            + [pltpu.VMEM((64,), jnp.int32)]
            + [pltpu.SemaphoreType.DMA] * 6
        ),
    )


def _sc_edge(a, c, et, rowp, colp, ss, npt, d):
    return _sc_edge_fn(npt, d)(a, c, et, rowp, colp, ss)


def _et_body(ea_ref, wee_ref, et_ref):
    et_ref[...] = jnp.dot(ea_ref[...], wee_ref[...],
                          preferred_element_type=jnp.float32)


def _stage_p_body(x_ref, oh_ref, u_ref, wea_ref, wec_ref, weu_ref, be_ref,
                  a_ref, c_ref):
    uw = jnp.dot(u_ref[...], weu_ref[...], preferred_element_type=jnp.float32)
    a_ref[...] = (jnp.dot(x_ref[...], wea_ref[...],
                          preferred_element_type=jnp.float32)
                  + jnp.dot(oh_ref[...], uw,
                            preferred_element_type=jnp.float32)
                  + be_ref[...])
    c_ref[...] = jnp.dot(x_ref[...], wec_ref[...],
                         preferred_element_type=jnp.float32)


def _stage_n_body(x_ref, agg_ref, oh_ref, u_ref, wnx_ref, wna_ref, wnu_ref,
                  bn_ref, wgu_ref, wgm_ref, bg_ref, xo_ref, uo_ref,
                  macc, cacc):
    i = pl.program_id(0)

    @pl.when(i == 0)
    def _():
        macc[...] = jnp.zeros_like(macc)
        cacc[...] = jnp.zeros_like(cacc)

    uw = jnp.dot(u_ref[...], wnu_ref[...], preferred_element_type=jnp.float32)
    xp = (jnp.dot(x_ref[...], wnx_ref[...],
                  preferred_element_type=jnp.float32)
          + jnp.dot(agg_ref[...], wna_ref[...],
                    preferred_element_type=jnp.float32)
          + jnp.dot(oh_ref[...], uw, preferred_element_type=jnp.float32)
          + bn_ref[...])
    xp = jnp.maximum(xp, 0.0)
    xo_ref[...] = xp
    oh = oh_ref[...]
    macc[...] += lax.dot_general(oh, xp, (((0,), (0,)), ((), ())),
                                 preferred_element_type=jnp.float32)
    ones = jnp.ones((oh.shape[0], 128), jnp.float32)
    cacc[...] += lax.dot_general(oh, ones, (((0,), (0,)), ((), ())),
                                 preferred_element_type=jnp.float32)

    @pl.when(i == pl.num_programs(0) - 1)
    def _():
        cnt = jnp.maximum(cacc[...][:, :1], 1.0)
        mean = macc[...] / cnt
        u2 = (jnp.dot(u_ref[...], wgu_ref[...],
                      preferred_element_type=jnp.float32)
              + jnp.dot(mean, wgm_ref[...],
                        preferred_element_type=jnp.float32)
              + bg_ref[...])
        uo_ref[...] = jnp.maximum(u2, 0.0)


def kernel(x, edge_attr, u, We, be, Wn, bn, Wg, bg, edge_index, batch):
    n, d = x.shape
    e, de = edge_attr.shape
    b = u.shape[0]
    npt = -(-n // (_NW * 8)) * 8
    n2 = npt * _NW

    row = edge_index[0]
    col = edge_index[1]

    order = jnp.argsort(col).astype(jnp.int32)
    col_s = jnp.take(col, order)
    row_s = jnp.take(row, order)
    bounds = jnp.arange(_NW + 1, dtype=jnp.int32) * npt
    ss = jnp.searchsorted(col_s, bounds).astype(jnp.int32)
    ss = jnp.pad(ss, (0, 64 - (_NW + 1)), constant_values=e)
    big = jnp.int32(1 << 22)
    pad = _EB + _L
    ept = -(-e // (_NW * _RB)) * _RB
    e2r = ept * _NW
    colp = jnp.concatenate([col_s, jnp.full((pad,), big, jnp.int32)])
    rowp = jnp.concatenate([row_s, jnp.zeros((pad,), jnp.int32)])
    ordp = jnp.concatenate([order, jnp.zeros((e2r - e,), jnp.int32)])
    oh = (batch[:, None] == jnp.arange(b, dtype=batch.dtype)[None, :])
    oh = oh.astype(jnp.float32)

    wea = We[:d]
    wec = We[d:2 * d]
    wee = We[2 * d:2 * d + de]
    weu = We[2 * d + de:]
    wnx = Wn[:d]
    wna = Wn[d:2 * d]
    wnu = Wn[2 * d:]
    wgu = Wg[:d]
    wgm = Wg[d:]
    be2 = be.reshape(1, d)
    bn2 = bn.reshape(1, d)
    bg2 = bg.reshape(1, d)

    dep = 128
    ea_pad = jnp.pad(edge_attr, ((0, 0), (0, dep - de)))
    wee_pad = jnp.pad(wee, ((0, dep - de), (0, 0)))
    ea_s = _sc_rows_fn(e2r, dep)(ea_pad, ordp)
    bep = e2r // 48
    et = pl.pallas_call(
        _et_body,
        grid=(48,),
        in_specs=[
            pl.BlockSpec((bep, dep), lambda i: (i, 0)),
            pl.BlockSpec((dep, d), lambda i: (0, 0)),
        ],
        out_specs=pl.BlockSpec((bep, d), lambda i: (i, 0)),
        out_shape=jax.ShapeDtypeStruct((e2r, d), jnp.float32),
    )(ea_s, wee_pad)

    bn_rows = 2000
    grid_n = n // bn_rows
    full = lambda i: (0, 0)

    stage_p = pl.pallas_call(
        _stage_p_body,
        grid=(grid_n,),
        in_specs=[
            pl.BlockSpec((bn_rows, d), lambda i: (i, 0)),
            pl.BlockSpec((bn_rows, b), lambda i: (i, 0)),
            pl.BlockSpec((b, d), full),
            pl.BlockSpec((d, d), full),
            pl.BlockSpec((d, d), full),
            pl.BlockSpec((d, d), full),
            pl.BlockSpec((1, d), full),
        ],
        out_specs=[
            pl.BlockSpec((bn_rows, d), lambda i: (i, 0)),
            pl.BlockSpec((bn_rows, d), lambda i: (i, 0)),
        ],
        out_shape=[
            jax.ShapeDtypeStruct((n, d), jnp.float32),
            jax.ShapeDtypeStruct((n, d), jnp.float32),
        ],
    )

    stage_n = pl.pallas_call(
        _stage_n_body,
        grid=(grid_n,),
        in_specs=[
            pl.BlockSpec((bn_rows, d), lambda i: (i, 0)),
            pl.BlockSpec((bn_rows, d), lambda i: (i, 0)),
            pl.BlockSpec((bn_rows, b), lambda i: (i, 0)),
            pl.BlockSpec((b, d), full),
            pl.BlockSpec((d, d), full),
            pl.BlockSpec((d, d), full),
            pl.BlockSpec((d, d), full),
            pl.BlockSpec((1, d), full),
            pl.BlockSpec((d, d), full),
            pl.BlockSpec((d, d), full),
            pl.BlockSpec((1, d), full),
        ],
        out_specs=[
            pl.BlockSpec((bn_rows, d), lambda i: (i, 0)),
            pl.BlockSpec((b, d), full),
        ],
        out_shape=[
            jax.ShapeDtypeStruct((n, d), jnp.float32),
            jax.ShapeDtypeStruct((b, d), jnp.float32),
        ],
        scratch_shapes=[
            pltpu.VMEM((b, d), jnp.float32),
            pltpu.VMEM((b, 128), jnp.float32),
        ],
    )

    outs = []
    steps = 4
    for _ in range(steps):
        a, c = stage_p(x, oh, u, wea, wec, weu, be2)
        agg = _sc_edge(a, c, et, rowp, colp, ss, npt, d)[:n]
        x, u = stage_n(x, agg, oh, u, wnx, wna, wnu, bn2, wgu, wgm, bg2)
        outs.append(u[:, None, :])
    return jnp.concatenate(outs, axis=1)
